# Initial kernel scaffold; baseline (speedup 1.0000x reference)
import jax
import jax.numpy as jnp
from jax.experimental import pallas as pl


def kernel(x, edge_index, edge_attr, batch, params):
    raise NotImplementedError("write your pallas kernel here")


# speedup vs baseline: 3.8584x; 3.8584x over previous
import functools

import jax
import jax.numpy as jnp
from jax import lax
from jax.experimental import pallas as pl
from jax.experimental.pallas import tpu as pltpu
from jax.experimental.pallas import tpu_sc as plsc

N_N = 10000
N_E = 320000
N_G = 64
D = 128

NC, NS = 2, 16
NW = NC * NS
EPT = N_E // NW
CK = 80
NCH = EPT // CK
N_NP = 10240
RPW = N_NP // NS
ZR = 40

BN = 2000
BE = 4000

_f32 = jnp.float32


def _zero_rows(zb_v, width):
    zf = jnp.zeros((16,), _f32)

    def _zrow(r, carry):
        for j in range(width // 16):
            zb_v[r, pl.ds(j * 16, 16)] = zf
        return carry

    lax.fori_loop(0, ZR, _zrow, 0)


def _gated_agg_body(h_hbm, g_hbm, src_hbm, dst_hbm, out_hbm,
                    src_v, dst_v, rows_v, gat_v, zb_v, acc_sh, sem):
    c = lax.axis_index("c")
    s = lax.axis_index("s")
    tid = c * NS + s
    eb = tid * EPT
    pltpu.sync_copy(src_hbm.at[pl.ds(eb, EPT)], src_v)
    pltpu.sync_copy(dst_hbm.at[pl.ds(eb, EPT)], dst_v)
    _zero_rows(zb_v, D)
    for t in range(RPW // ZR):
        pltpu.sync_copy(zb_v, acc_sh.at[pl.ds(s * RPW + t * ZR, ZR)])
    plsc.subcore_barrier()

    def _chunk(k, carry):
        off = k * CK
        pltpu.async_copy(h_hbm.at[src_v.at[pl.ds(off, CK)]], rows_v, sem).wait()
        pltpu.sync_copy(g_hbm.at[pl.ds(eb + off, CK)], gat_v)

        def _mrow(r, carry2):
            for j in range(D // 16):
                sl = (r, pl.ds(j * 16, 16))
                rows_v[sl] = rows_v[sl] * gat_v[sl]
            return carry2

        lax.fori_loop(0, CK, _mrow, 0)
        pltpu.sync_copy(rows_v, acc_sh.at[dst_v.at[pl.ds(off, CK)]], add=True)
        return carry

    lax.fori_loop(0, NCH, _chunk, 0)
    plsc.subcore_barrier()
    rb = s * RPW
    pltpu.sync_copy(acc_sh.at[pl.ds(rb, RPW)], out_hbm.at[c, pl.ds(rb, RPW)])


@functools.lru_cache(maxsize=None)
def _gated_agg_kernel():
    mesh = plsc.VectorSubcoreMesh(
        core_axis_name="c", subcore_axis_name="s",
        num_cores=NC, num_subcores=NS)
    return pl.kernel(
        _gated_agg_body,
        out_type=jax.ShapeDtypeStruct((NC, N_NP, D), _f32),
        mesh=mesh,
        scratch_types=[
            pltpu.VMEM((EPT,), jnp.int32),
            pltpu.VMEM((EPT,), jnp.int32),
            pltpu.VMEM((CK, D), _f32),
            pltpu.VMEM((CK, D), _f32),
            pltpu.VMEM((ZR, D), _f32),
            pltpu.VMEM_SHARED((N_NP, D), _f32),
            pltpu.SemaphoreType.DMA,
        ],
    )


def _gated_agg(h, g, src, dst):
    return _gated_agg_kernel()(h, g, src, dst)


def _mean_agg_body(h_hbm, src_hbm, dst_hbm, out_hbm,
                   src_v, dst_v, rows_v, zb_v, acc_sh, sem):
    c = lax.axis_index("c")
    s = lax.axis_index("s")
    tid = c * NS + s
    eb = tid * EPT
    pltpu.sync_copy(src_hbm.at[pl.ds(eb, EPT)], src_v)
    pltpu.sync_copy(dst_hbm.at[pl.ds(eb, EPT)], dst_v)
    _zero_rows(zb_v, D)
    for t in range(RPW // ZR):
        pltpu.sync_copy(zb_v, acc_sh.at[pl.ds(s * RPW + t * ZR, ZR)])
    plsc.subcore_barrier()

    def _chunk(k, carry):
        off = k * CK
        pltpu.async_copy(h_hbm.at[src_v.at[pl.ds(off, CK)]], rows_v, sem).wait()
        pltpu.sync_copy(rows_v, acc_sh.at[dst_v.at[pl.ds(off, CK)]], add=True)
        return carry

    lax.fori_loop(0, NCH, _chunk, 0)
    plsc.subcore_barrier()
    rb = s * RPW
    pltpu.sync_copy(acc_sh.at[pl.ds(rb, RPW)], out_hbm.at[c, pl.ds(rb, RPW)])


@functools.lru_cache(maxsize=None)
def _mean_agg_kernel():
    mesh = plsc.VectorSubcoreMesh(
        core_axis_name="c", subcore_axis_name="s",
        num_cores=NC, num_subcores=NS)
    return pl.kernel(
        _mean_agg_body,
        out_type=jax.ShapeDtypeStruct((NC, N_NP, D), _f32),
        mesh=mesh,
        scratch_types=[
            pltpu.VMEM((EPT,), jnp.int32),
            pltpu.VMEM((EPT,), jnp.int32),
            pltpu.VMEM((CK, D), _f32),
            pltpu.VMEM((ZR, D), _f32),
            pltpu.VMEM_SHARED((N_NP, D), _f32),
            pltpu.SemaphoreType.DMA,
        ],
    )


def _mean_agg(h, src, dst):
    return _mean_agg_kernel()(h, src, dst)


def _ln(v, g, b):
    m = jnp.mean(v, axis=-1, keepdims=True)
    var = jnp.mean((v - m) ** 2, axis=-1, keepdims=True)
    return (v - m) / jnp.sqrt(var + 1e-5) * g + b


def _enc_body(xb, wi, wp, wx, wo, wt, cw1, cb1, mtt, ball, g, be, h0_out):
    xv = xb[...]

    def oh(col, width):
        v = xv[:, col:col + 1].astype(jnp.int32)
        return (v == lax.broadcasted_iota(jnp.int32, (BN, width), 1)).astype(_f32)

    acc = oh(0, 16) @ wi[...]
    acc += oh(1, 128) @ wp[...]
    acc += oh(6, 16) @ wx[...]
    acc += oh(7, 16) @ wo[...]
    acc += oh(8, 16) @ wt[...]
    a1 = jnp.maximum(xv[:, 2:10] @ cw1[...] + cb1[...], 0.0)
    acc += a1 @ mtt[...]
    acc += ball[...]
    h0_out[...] = jnp.maximum(_ln(acc, g[...], be[...]), 0.0)


def _gates_body(ea, w1t, b1, w2t, b2, gwt, gb, *gouts):
    a = ea[...]
    e1 = jnp.maximum(a @ w1t[...] + b1[...], 0.0)
    ee = e1 @ w2t[...] + b2[...]
    for l in range(8):
        z = ee @ gwt[l] + gb[l]
        gouts[l][...] = jax.nn.sigmoid(z)


def _dense1_body(h, part, wc1t, wc2t, cb, projt, pb, hn_out, hp_out):
    pr = part[...]
    agg = pr[0] + pr[1]
    hn = h[...] @ wc1t[...] + agg @ wc2t[...] + cb[...]
    hn_out[...] = hn
    hp_out[...] = jnp.maximum(hn @ projt[...] + pb[...], 0.0)


def _dense2_body(part, hn, ident, degp, llt, llb, lrt, lng, lnb, h_out):
    dp = degp[...]
    deg = jnp.clip(dp[0, :, 0:1] + dp[1, :, 0:1], 1.0, None)
    pr = part[...]
    mean = (pr[0] + pr[1]) / deg
    out = mean @ llt[...] + llb[...] + hn[...] @ lrt[...]
    nrm = jnp.sqrt(jnp.sum(out * out, axis=-1, keepdims=True))
    out = out / jnp.clip(nrm, 1e-12, None)
    out = _ln(out, lng[...], lnb[...])
    h_out[...] = jnp.maximum(out + ident[...], 0.0)


def _readout_body(h, bcol, a1t, a1b, a2t, a2b, f1at, f1bt, f1ct, fb1,
                  fpg, fpbe, f2t, fb2, z_out, xmax_v, gmax_v):
    hv = h[...]
    bv = bcol[...]
    gate = jnp.tanh(hv @ a1t[...] + a1b[...]) @ a2t[...] + a2b[...]

    def _seg(gi, carry):
        mb = bv == gi
        hm = jnp.where(mb, hv, -3e38)
        xmax_v[pl.ds(gi, 1), :] = jnp.max(hm, axis=0, keepdims=True)
        gm = jnp.max(jnp.where(mb, gate, -3e38), axis=0, keepdims=True)
        gmax_v[pl.ds(gi, 1), :] = jnp.broadcast_to(gm, (1, D))
        return carry

    lax.fori_loop(0, N_G, _seg, 0)
    x_max = xmax_v[...]
    gmax = gmax_v[...][:, 0:1]

    maskt = (bv == lax.broadcasted_iota(jnp.int32, (N_N, N_G), 1)).astype(_f32)
    dn = (((0,), (0,)), ((), ()))
    counts = lax.dot_general(maskt, jnp.ones((N_N, 1), _f32), dn)
    gmax_pn = maskt @ gmax
    ge = jnp.exp(gate - gmax_pn)
    gsum = lax.dot_general(maskt, ge, dn)
    gsum_pn = maskt @ gsum
    alpha = ge / (gsum_pn + 1e-16)
    x_att = lax.dot_general(maskt, alpha * hv, dn)
    x_mean = lax.dot_general(maskt, hv, dn) / jnp.clip(counts, 1.0, None)

    z1 = x_att @ f1at[...] + x_mean @ f1bt[...] + x_max @ f1ct[...] + fb1[...]
    z1 = jnp.maximum(_ln(z1, fpg[...], fpbe[...]), 0.0)
    z2 = z1 @ f2t[...] + fb2[...]
    z_out[...] = jnp.maximum(z2, 0.0) + jnp.log(1.0 + jnp.exp(-jnp.abs(z2)))


def _full_spec(shape):
    nd = len(shape)
    return pl.BlockSpec(shape, lambda i, _nd=nd: (0,) * _nd)


def _run_enc(xp, wi, wp, wx, wo, wt, cw1, cb1, mtt, ball, g, be):
    grid = (N_N // BN,)
    return pl.pallas_call(
        _enc_body,
        grid=grid,
        in_specs=[
            pl.BlockSpec((BN, 16), lambda i: (i, 0)),
            _full_spec((16, D)), _full_spec((128, D)), _full_spec((16, D)),
            _full_spec((16, D)), _full_spec((16, D)), _full_spec((8, 128)),
            _full_spec((1, 128)), _full_spec((128, 128)), _full_spec((1, D)),
            _full_spec((1, D)), _full_spec((1, D)),
        ],
        out_specs=pl.BlockSpec((BN, D), lambda i: (i, 0)),
        out_shape=jax.ShapeDtypeStruct((N_N, D), _f32),
    )(xp, wi, wp, wx, wo, wt, cw1, cb1, mtt, ball, g, be)


def _run_gates(ea8, w1t, b1, w2t, b2, gwt, gb):
    grid = (N_E // BE,)
    sds = jax.ShapeDtypeStruct((N_E, D), _f32)
    return pl.pallas_call(
        _gates_body,
        grid=grid,
        in_specs=[
            pl.BlockSpec((BE, 8), lambda i: (i, 0)),
            _full_spec((8, 32)), _full_spec((1, 32)), _full_spec((32, 32)),
            _full_spec((1, 32)), _full_spec((8, 32, 128)), _full_spec((8, 1, 128)),
        ],
        out_specs=[pl.BlockSpec((BE, D), lambda i: (i, 0))] * 8,
        out_shape=[sds] * 8,
    )(ea8, w1t, b1, w2t, b2, gwt, gb)


def _run_dense1(h, part, wc1t, wc2t, cb, projt, pb):
    grid = (N_N // BN,)
    sds = jax.ShapeDtypeStruct((N_N, D), _f32)
    return pl.pallas_call(
        _dense1_body,
        grid=grid,
        in_specs=[
            pl.BlockSpec((BN, D), lambda i: (i, 0)),
            pl.BlockSpec((NC, BN, D), lambda i: (0, i, 0)),
            _full_spec((D, D)), _full_spec((D, D)), _full_spec((1, D)),
            _full_spec((D, D)), _full_spec((1, D)),
        ],
        out_specs=[pl.BlockSpec((BN, D), lambda i: (i, 0))] * 2,
        out_shape=[sds] * 2,
    )(h, part, wc1t, wc2t, cb, projt, pb)


def _run_dense2(part, hn, ident, degp, llt, llb, lrt, lng, lnb):
    grid = (N_N // BN,)
    return pl.pallas_call(
        _dense2_body,
        grid=grid,
        in_specs=[
            pl.BlockSpec((NC, BN, D), lambda i: (0, i, 0)),
            pl.BlockSpec((BN, D), lambda i: (i, 0)),
            pl.BlockSpec((BN, D), lambda i: (i, 0)),
            pl.BlockSpec((NC, BN, D), lambda i: (0, i, 0)),
            _full_spec((D, D)), _full_spec((1, D)), _full_spec((D, D)),
            _full_spec((1, D)), _full_spec((1, D)),
        ],
        out_specs=pl.BlockSpec((BN, D), lambda i: (i, 0)),
        out_shape=jax.ShapeDtypeStruct((N_N, D), _f32),
    )(part, hn, ident, degp, llt, llb, lrt, lng, lnb)


def _run_readout(h, bcol, a1t, a1b, a2t, a2b, f1at, f1bt, f1ct, fb1,
                 fpg, fpbe, f2t, fb2):
    return pl.pallas_call(
        _readout_body,
        grid=(1,),
        in_specs=[
            _full_spec((N_N, D)), _full_spec((N_N, 1)),
            _full_spec((D, N_G)), _full_spec((1, N_G)), _full_spec((N_G, 1)),
            _full_spec((1, 1)),
            _full_spec((D, D)), _full_spec((D, D)), _full_spec((D, D)),
            _full_spec((1, D)), _full_spec((1, D)), _full_spec((1, D)),
            _full_spec((D, N_G)), _full_spec((1, N_G)),
        ],
        out_specs=pl.BlockSpec((N_G, N_G), lambda i: (0, 0)),
        out_shape=jax.ShapeDtypeStruct((N_G, N_G), _f32),
        scratch_shapes=[
            pltpu.VMEM((N_G, D), _f32),
            pltpu.VMEM((N_G, D), _f32),
        ],
    )(h, bcol, a1t, a1b, a2t, a2b, f1at, f1bt, f1ct, fb1, fpg, fpbe, f2t, fb2)


def kernel(x, edge_index, edge_attr, batch, params):
    p = params
    x = x.astype(_f32)
    src = edge_index[0].astype(jnp.int32)
    dst = edge_index[1].astype(jnp.int32)
    bcol = batch.reshape(N_N, 1).astype(jnp.int32)

    ftw = p['ft_W']
    wi = p['inst_emb'] @ ftw[:, 0:8].T
    wp = p['pitch_emb'] @ ftw[:, 8:24].T
    wx = p['index_emb'] @ ftw[:, 24:32].T
    wo = jnp.pad(p['oct_emb'] @ ftw[:, 32:36].T, ((0, 4), (0, 0)))
    wt = jnp.pad(p['ts_emb'] @ ftw[:, 36:40].T, ((0, 1), (0, 0)))
    cw1 = jnp.pad(p['ce_W1'].T, ((0, 4), (0, 0)))
    cb1 = p['ce_b1'].reshape(1, 128)
    mtt = (ftw[:, 40:104] @ p['ce_W2']).T
    ball = (p['ft_b'] + p['ce_b2'] @ ftw[:, 40:104].T).reshape(1, D)

    xp = jnp.pad(x, ((0, 0), (0, 7)))
    h = _run_enc(xp, wi, wp, wx, wo, wt, cw1, cb1, mtt, ball,
                 p['ft_g'].reshape(1, D), p['ft_be'].reshape(1, D))

    ea8 = jnp.pad(edge_attr.astype(_f32), ((0, 0), (0, 7)))
    w1t = jnp.pad(p['ee_W1'].T, ((0, 7), (0, 0)))
    b1 = p['ee_b1'].reshape(1, 32)
    w2t = p['ee_W2'].T
    b2 = p['ee_b2'].reshape(1, 32)
    gwt = jnp.stack([l['gate_W'].T for l in p['layers']])
    gb = jnp.stack([l['gate_b'].reshape(1, D) for l in p['layers']])
    gates = _run_gates(ea8, w1t, b1, w2t, b2, gwt, gb)

    degp = _mean_agg(jnp.ones((N_N, D), _f32), src, dst)

    for l in range(8):
        lp = p['layers'][l]
        part = _gated_agg(h, gates[l], src, dst)
        hn, hp = _run_dense1(
            h, part,
            lp['comb_W'][:, :D].T, lp['comb_W'][:, D:].T,
            lp['comb_b'].reshape(1, D),
            lp['proj_W'].T, lp['proj_b'].reshape(1, D))
        part2 = _mean_agg(hp, src, dst)
        h = _run_dense2(
            part2, hn, h, degp,
            lp['lin_l_W'].T, lp['lin_l_b'].reshape(1, D),
            lp['lin_r_W'].T,
            lp['ln_g'].reshape(1, D), lp['ln_b'].reshape(1, D))

    z = _run_readout(
        h, bcol,
        p['att_W1'].T, p['att_b1'].reshape(1, N_G),
        p['att_W2'].T, p['att_b2'].reshape(1, 1),
        p['fp_W1'][:, 0:D].T, p['fp_W1'][:, D:2 * D].T, p['fp_W1'][:, 2 * D:].T,
        p['fp_b1'].reshape(1, D),
        p['fp_g'].reshape(1, D), p['fp_be'].reshape(1, D),
        p['fp_W2'].T, p['fp_b2'].reshape(1, N_G))
    return z


# speedup vs baseline: 5.7803x; 1.4981x over previous
import functools

import jax
import jax.numpy as jnp
from jax import lax
from jax.experimental import pallas as pl
from jax.experimental.pallas import tpu as pltpu
from jax.experimental.pallas import tpu_sc as plsc

N_N = 10000
N_E = 320000
N_G = 64
D = 128

NC, NS = 2, 16
NW = NC * NS
EPT = N_E // NW
CK = 40
NCH = EPT // CK
NPAIR = NCH // 2
N_NP = 10240
RPW = N_NP // NS
ZR = 40

BN = 2000
BE = 4000

_f32 = jnp.float32


def _zero_rows(zb_v, width):
    zf = jnp.zeros((16,), _f32)

    def _zrow(r, carry):
        for j in range(width // 16):
            zb_v[r, pl.ds(j * 16, 16)] = zf
        return carry

    lax.fori_loop(0, ZR, _zrow, 0)


def _gated_agg_body(h_hbm, g_hbm, src_hbm, dst_hbm, out_hbm,
                    src_v, dst_v, rows_a, rows_b, gat_a, gat_b, zb_v, acc_sh,
                    sem_ga, sem_gga, sem_gb, sem_ggb):
    c = lax.axis_index("c")
    s = lax.axis_index("s")
    tid = c * NS + s
    eb = tid * EPT
    pltpu.sync_copy(src_hbm.at[pl.ds(eb, EPT)], src_v)
    pltpu.sync_copy(dst_hbm.at[pl.ds(eb, EPT)], dst_v)
    _zero_rows(zb_v, D)
    for t in range(RPW // ZR):
        pltpu.sync_copy(zb_v, acc_sh.at[pl.ds(s * RPW + t * ZR, ZR)])
    plsc.subcore_barrier()

    def _issue(off, rows, gat, sg, sgg):
        pltpu.async_copy(h_hbm.at[src_v.at[pl.ds(off, CK)]], rows, sg)
        pltpu.async_copy(g_hbm.at[pl.ds(eb + off, CK)], gat, sgg)

    def _wait(off, rows, gat, sg, sgg):
        pltpu.make_async_copy(h_hbm.at[src_v.at[pl.ds(off, CK)]], rows, sg).wait()
        pltpu.make_async_copy(g_hbm.at[pl.ds(eb + off, CK)], gat, sgg).wait()

    def _mult_scat(off, rows, gat):
        def _mrow(r, carry2):
            for j in range(D // 16):
                sl = (r, pl.ds(j * 16, 16))
                gat[sl] = rows[sl] * gat[sl]
            return carry2

        lax.fori_loop(0, CK, _mrow, 0)
        pltpu.sync_copy(gat, acc_sh.at[dst_v.at[pl.ds(off, CK)]], add=True)

    _issue(0, rows_a, gat_a, sem_ga, sem_gga)

    def _pair(i, carry):
        off_e = i * (2 * CK)
        off_o = off_e + CK
        _issue(off_o, rows_b, gat_b, sem_gb, sem_ggb)
        _wait(off_e, rows_a, gat_a, sem_ga, sem_gga)
        _mult_scat(off_e, rows_a, gat_a)

        @pl.when(i < NPAIR - 1)
        def _():
            _issue(off_e + 2 * CK, rows_a, gat_a, sem_ga, sem_gga)

        _wait(off_o, rows_b, gat_b, sem_gb, sem_ggb)
        _mult_scat(off_o, rows_b, gat_b)
        return carry

    lax.fori_loop(0, NPAIR, _pair, 0)
    plsc.subcore_barrier()
    rb = s * RPW
    pltpu.sync_copy(acc_sh.at[pl.ds(rb, RPW)], out_hbm.at[c, pl.ds(rb, RPW)])


@functools.lru_cache(maxsize=None)
def _gated_agg_kernel():
    mesh = plsc.VectorSubcoreMesh(
        core_axis_name="c", subcore_axis_name="s",
        num_cores=NC, num_subcores=NS)
    return pl.kernel(
        _gated_agg_body,
        out_type=jax.ShapeDtypeStruct((NC, N_NP, D), _f32),
        mesh=mesh,
        scratch_types=[
            pltpu.VMEM((EPT,), jnp.int32),
            pltpu.VMEM((EPT,), jnp.int32),
            pltpu.VMEM((CK, D), _f32),
            pltpu.VMEM((CK, D), _f32),
            pltpu.VMEM((CK, D), _f32),
            pltpu.VMEM((CK, D), _f32),
            pltpu.VMEM((ZR, D), _f32),
            pltpu.VMEM_SHARED((N_NP, D), _f32),
            pltpu.SemaphoreType.DMA,
            pltpu.SemaphoreType.DMA,
            pltpu.SemaphoreType.DMA,
            pltpu.SemaphoreType.DMA,
        ],
    )


def _gated_agg(h, g, src, dst):
    return _gated_agg_kernel()(h, g, src, dst)


def _mean_agg_body(h_hbm, src_hbm, dst_hbm, out_hbm,
                   src_v, dst_v, rows_a, rows_b, zb_v, acc_sh,
                   sem_a, sem_b):
    c = lax.axis_index("c")
    s = lax.axis_index("s")
    tid = c * NS + s
    eb = tid * EPT
    pltpu.sync_copy(src_hbm.at[pl.ds(eb, EPT)], src_v)
    pltpu.sync_copy(dst_hbm.at[pl.ds(eb, EPT)], dst_v)
    _zero_rows(zb_v, D)
    for t in range(RPW // ZR):
        pltpu.sync_copy(zb_v, acc_sh.at[pl.ds(s * RPW + t * ZR, ZR)])
    plsc.subcore_barrier()

    def _issue(off, rows, sg):
        pltpu.async_copy(h_hbm.at[src_v.at[pl.ds(off, CK)]], rows, sg)

    def _wait(off, rows, sg):
        pltpu.make_async_copy(h_hbm.at[src_v.at[pl.ds(off, CK)]], rows, sg).wait()

    def _scat(off, rows):
        pltpu.sync_copy(rows, acc_sh.at[dst_v.at[pl.ds(off, CK)]], add=True)

    _issue(0, rows_a, sem_a)

    def _pair(i, carry):
        off_e = i * (2 * CK)
        off_o = off_e + CK
        _issue(off_o, rows_b, sem_b)
        _wait(off_e, rows_a, sem_a)
        _scat(off_e, rows_a)

        @pl.when(i < NPAIR - 1)
        def _():
            _issue(off_e + 2 * CK, rows_a, sem_a)

        _wait(off_o, rows_b, sem_b)
        _scat(off_o, rows_b)
        return carry

    lax.fori_loop(0, NPAIR, _pair, 0)
    plsc.subcore_barrier()
    rb = s * RPW
    pltpu.sync_copy(acc_sh.at[pl.ds(rb, RPW)], out_hbm.at[c, pl.ds(rb, RPW)])


@functools.lru_cache(maxsize=None)
def _mean_agg_kernel():
    mesh = plsc.VectorSubcoreMesh(
        core_axis_name="c", subcore_axis_name="s",
        num_cores=NC, num_subcores=NS)
    return pl.kernel(
        _mean_agg_body,
        out_type=jax.ShapeDtypeStruct((NC, N_NP, D), _f32),
        mesh=mesh,
        scratch_types=[
            pltpu.VMEM((EPT,), jnp.int32),
            pltpu.VMEM((EPT,), jnp.int32),
            pltpu.VMEM((CK, D), _f32),
            pltpu.VMEM((CK, D), _f32),
            pltpu.VMEM((ZR, D), _f32),
            pltpu.VMEM_SHARED((N_NP, D), _f32),
            pltpu.SemaphoreType.DMA,
            pltpu.SemaphoreType.DMA,
        ],
    )


def _mean_agg(h, src, dst):
    return _mean_agg_kernel()(h, src, dst)


def _ln(v, g, b):
    m = jnp.mean(v, axis=-1, keepdims=True)
    var = jnp.mean((v - m) ** 2, axis=-1, keepdims=True)
    return (v - m) / jnp.sqrt(var + 1e-5) * g + b


def _enc_body(xb, wi, wp, wx, wo, wt, cw1, cb1, mtt, ball, g, be, h0_out):
    xv = xb[...]

    def oh(col, width):
        v = xv[:, col:col + 1].astype(jnp.int32)
        return (v == lax.broadcasted_iota(jnp.int32, (BN, width), 1)).astype(_f32)

    acc = oh(0, 16) @ wi[...]
    acc += oh(1, 128) @ wp[...]
    acc += oh(6, 16) @ wx[...]
    acc += oh(7, 16) @ wo[...]
    acc += oh(8, 16) @ wt[...]
    a1 = jnp.maximum(xv[:, 2:10] @ cw1[...] + cb1[...], 0.0)
    acc += a1 @ mtt[...]
    acc += ball[...]
    h0_out[...] = jnp.maximum(_ln(acc, g[...], be[...]), 0.0)


def _gates_body(ea, w1t, b1, w2t, b2, gwt, gb, *gouts):
    a = ea[...]
    e1 = jnp.maximum(a @ w1t[...] + b1[...], 0.0)
    ee = e1 @ w2t[...] + b2[...]
    for l in range(8):
        z = ee @ gwt[l] + gb[l]
        gouts[l][...] = jax.nn.sigmoid(z)


def _dense1_body(h, part, wc1t, wc2t, cb, projt, pb, hn_out, hp_out):
    pr = part[...]
    agg = pr[0] + pr[1]
    hn = h[...] @ wc1t[...] + agg @ wc2t[...] + cb[...]
    hn_out[...] = hn
    hp_out[...] = jnp.maximum(hn @ projt[...] + pb[...], 0.0)


def _dense2_body(part, hn, ident, degp, llt, llb, lrt, lng, lnb, h_out):
    dp = degp[...]
    deg = jnp.clip(dp[0, :, 0:1] + dp[1, :, 0:1], 1.0, None)
    pr = part[...]
    mean = (pr[0] + pr[1]) / deg
    out = mean @ llt[...] + llb[...] + hn[...] @ lrt[...]
    nrm = jnp.sqrt(jnp.sum(out * out, axis=-1, keepdims=True))
    out = out / jnp.clip(nrm, 1e-12, None)
    out = _ln(out, lng[...], lnb[...])
    h_out[...] = jnp.maximum(out + ident[...], 0.0)


def _readout_body(h, bcol, a1t, a1b, a2t, a2b, f1at, f1bt, f1ct, fb1,
                  fpg, fpbe, f2t, fb2, z_out, xmax_v, gmax_v):
    hv = h[...]
    bv = bcol[...]
    gate = jnp.tanh(hv @ a1t[...] + a1b[...]) @ a2t[...] + a2b[...]

    def _seg(gi, carry):
        mb = bv == gi
        hm = jnp.where(mb, hv, -3e38)
        xmax_v[pl.ds(gi, 1), :] = jnp.max(hm, axis=0, keepdims=True)
        gm = jnp.max(jnp.where(mb, gate, -3e38), axis=0, keepdims=True)
        gmax_v[pl.ds(gi, 1), :] = jnp.broadcast_to(gm, (1, D))
        return carry

    lax.fori_loop(0, N_G, _seg, 0)
    x_max = xmax_v[...]
    gmax = gmax_v[...][:, 0:1]

    maskt = (bv == lax.broadcasted_iota(jnp.int32, (N_N, N_G), 1)).astype(_f32)
    dn = (((0,), (0,)), ((), ()))
    counts = lax.dot_general(maskt, jnp.ones((N_N, 1), _f32), dn)
    gmax_pn = maskt @ gmax
    ge = jnp.exp(gate - gmax_pn)
    gsum = lax.dot_general(maskt, ge, dn)
    gsum_pn = maskt @ gsum
    alpha = ge / (gsum_pn + 1e-16)
    x_att = lax.dot_general(maskt, alpha * hv, dn)
    x_mean = lax.dot_general(maskt, hv, dn) / jnp.clip(counts, 1.0, None)

    z1 = x_att @ f1at[...] + x_mean @ f1bt[...] + x_max @ f1ct[...] + fb1[...]
    z1 = jnp.maximum(_ln(z1, fpg[...], fpbe[...]), 0.0)
    z2 = z1 @ f2t[...] + fb2[...]
    z_out[...] = jnp.maximum(z2, 0.0) + jnp.log(1.0 + jnp.exp(-jnp.abs(z2)))


def _full_spec(shape):
    nd = len(shape)
    return pl.BlockSpec(shape, lambda i, _nd=nd: (0,) * _nd)


def _run_enc(xp, wi, wp, wx, wo, wt, cw1, cb1, mtt, ball, g, be):
    grid = (N_N // BN,)
    return pl.pallas_call(
        _enc_body,
        grid=grid,
        in_specs=[
            pl.BlockSpec((BN, 16), lambda i: (i, 0)),
            _full_spec((16, D)), _full_spec((128, D)), _full_spec((16, D)),
            _full_spec((16, D)), _full_spec((16, D)), _full_spec((8, 128)),
            _full_spec((1, 128)), _full_spec((128, 128)), _full_spec((1, D)),
            _full_spec((1, D)), _full_spec((1, D)),
        ],
        out_specs=pl.BlockSpec((BN, D), lambda i: (i, 0)),
        out_shape=jax.ShapeDtypeStruct((N_N, D), _f32),
    )(xp, wi, wp, wx, wo, wt, cw1, cb1, mtt, ball, g, be)


def _run_gates(ea8, w1t, b1, w2t, b2, gwt, gb):
    grid = (N_E // BE,)
    sds = jax.ShapeDtypeStruct((N_E, D), _f32)
    return pl.pallas_call(
        _gates_body,
        grid=grid,
        in_specs=[
            pl.BlockSpec((BE, 8), lambda i: (i, 0)),
            _full_spec((8, 32)), _full_spec((1, 32)), _full_spec((32, 32)),
            _full_spec((1, 32)), _full_spec((8, 32, 128)), _full_spec((8, 1, 128)),
        ],
        out_specs=[pl.BlockSpec((BE, D), lambda i: (i, 0))] * 8,
        out_shape=[sds] * 8,
    )(ea8, w1t, b1, w2t, b2, gwt, gb)


def _run_dense1(h, part, wc1t, wc2t, cb, projt, pb):
    grid = (N_N // BN,)
    sds = jax.ShapeDtypeStruct((N_N, D), _f32)
    return pl.pallas_call(
        _dense1_body,
        grid=grid,
        in_specs=[
            pl.BlockSpec((BN, D), lambda i: (i, 0)),
            pl.BlockSpec((NC, BN, D), lambda i: (0, i, 0)),
            _full_spec((D, D)), _full_spec((D, D)), _full_spec((1, D)),
            _full_spec((D, D)), _full_spec((1, D)),
        ],
        out_specs=[pl.BlockSpec((BN, D), lambda i: (i, 0))] * 2,
        out_shape=[sds] * 2,
    )(h, part, wc1t, wc2t, cb, projt, pb)


def _run_dense2(part, hn, ident, degp, llt, llb, lrt, lng, lnb):
    grid = (N_N // BN,)
    return pl.pallas_call(
        _dense2_body,
        grid=grid,
        in_specs=[
            pl.BlockSpec((NC, BN, D), lambda i: (0, i, 0)),
            pl.BlockSpec((BN, D), lambda i: (i, 0)),
            pl.BlockSpec((BN, D), lambda i: (i, 0)),
            pl.BlockSpec((NC, BN, D), lambda i: (0, i, 0)),
            _full_spec((D, D)), _full_spec((1, D)), _full_spec((D, D)),
            _full_spec((1, D)), _full_spec((1, D)),
        ],
        out_specs=pl.BlockSpec((BN, D), lambda i: (i, 0)),
        out_shape=jax.ShapeDtypeStruct((N_N, D), _f32),
    )(part, hn, ident, degp, llt, llb, lrt, lng, lnb)


def _run_readout(h, bcol, a1t, a1b, a2t, a2b, f1at, f1bt, f1ct, fb1,
                 fpg, fpbe, f2t, fb2):
    return pl.pallas_call(
        _readout_body,
        grid=(1,),
        in_specs=[
            _full_spec((N_N, D)), _full_spec((N_N, 1)),
            _full_spec((D, N_G)), _full_spec((1, N_G)), _full_spec((N_G, 1)),
            _full_spec((1, 1)),
            _full_spec((D, D)), _full_spec((D, D)), _full_spec((D, D)),
            _full_spec((1, D)), _full_spec((1, D)), _full_spec((1, D)),
            _full_spec((D, N_G)), _full_spec((1, N_G)),
        ],
        out_specs=pl.BlockSpec((N_G, N_G), lambda i: (0, 0)),
        out_shape=jax.ShapeDtypeStruct((N_G, N_G), _f32),
        scratch_shapes=[
            pltpu.VMEM((N_G, D), _f32),
            pltpu.VMEM((N_G, D), _f32),
        ],
    )(h, bcol, a1t, a1b, a2t, a2b, f1at, f1bt, f1ct, fb1, fpg, fpbe, f2t, fb2)


def kernel(x, edge_index, edge_attr, batch, params):
    p = params
    x = x.astype(_f32)
    src = edge_index[0].astype(jnp.int32)
    dst = edge_index[1].astype(jnp.int32)
    bcol = batch.reshape(N_N, 1).astype(jnp.int32)

    ftw = p['ft_W']
    wi = p['inst_emb'] @ ftw[:, 0:8].T
    wp = p['pitch_emb'] @ ftw[:, 8:24].T
    wx = p['index_emb'] @ ftw[:, 24:32].T
    wo = jnp.pad(p['oct_emb'] @ ftw[:, 32:36].T, ((0, 4), (0, 0)))
    wt = jnp.pad(p['ts_emb'] @ ftw[:, 36:40].T, ((0, 1), (0, 0)))
    cw1 = jnp.pad(p['ce_W1'].T, ((0, 4), (0, 0)))
    cb1 = p['ce_b1'].reshape(1, 128)
    mtt = (ftw[:, 40:104] @ p['ce_W2']).T
    ball = (p['ft_b'] + p['ce_b2'] @ ftw[:, 40:104].T).reshape(1, D)

    xp = jnp.pad(x, ((0, 0), (0, 7)))
    h = _run_enc(xp, wi, wp, wx, wo, wt, cw1, cb1, mtt, ball,
                 p['ft_g'].reshape(1, D), p['ft_be'].reshape(1, D))

    ea8 = jnp.pad(edge_attr.astype(_f32), ((0, 0), (0, 7)))
    w1t = jnp.pad(p['ee_W1'].T, ((0, 7), (0, 0)))
    b1 = p['ee_b1'].reshape(1, 32)
    w2t = p['ee_W2'].T
    b2 = p['ee_b2'].reshape(1, 32)
    gwt = jnp.stack([l['gate_W'].T for l in p['layers']])
    gb = jnp.stack([l['gate_b'].reshape(1, D) for l in p['layers']])
    gates = _run_gates(ea8, w1t, b1, w2t, b2, gwt, gb)

    degp = _mean_agg(jnp.ones((N_N, D), _f32), src, dst)

    for l in range(8):
        lp = p['layers'][l]
        part = _gated_agg(h, gates[l], src, dst)
        hn, hp = _run_dense1(
            h, part,
            lp['comb_W'][:, :D].T, lp['comb_W'][:, D:].T,
            lp['comb_b'].reshape(1, D),
            lp['proj_W'].T, lp['proj_b'].reshape(1, D))
        part2 = _mean_agg(hp, src, dst)
        h = _run_dense2(
            part2, hn, h, degp,
            lp['lin_l_W'].T, lp['lin_l_b'].reshape(1, D),
            lp['lin_r_W'].T,
            lp['ln_g'].reshape(1, D), lp['ln_b'].reshape(1, D))

    z = _run_readout(
        h, bcol,
        p['att_W1'].T, p['att_b1'].reshape(1, N_G),
        p['att_W2'].T, p['att_b2'].reshape(1, 1),
        p['fp_W1'][:, 0:D].T, p['fp_W1'][:, D:2 * D].T, p['fp_W1'][:, 2 * D:].T,
        p['fp_b1'].reshape(1, D),
        p['fp_g'].reshape(1, D), p['fp_be'].reshape(1, D),
        p['fp_W2'].T, p['fp_b2'].reshape(1, N_G))
    return z
